# baseline (speedup 1.0000x reference)
import functools

import jax
import jax.numpy as jnp
from jax import lax
from jax.experimental import pallas as pl
from jax.experimental.pallas import tpu as pltpu
from jax.experimental.pallas import tpu_sc as plsc

N = 10000
E = 160000
IN_FEATS = 256
N_HIDDEN = 512
N_CLASSES = 256

NC = 2
NS = 16
NW = NC * NS
EPW = E // NW
CHUNK = 128
NCH = -(-EPW // CHUNK)
EPW_PAD = NCH * CHUNK
PADE = EPW_PAD - EPW
NPAD = 12800
STRIPE = NPAD // NS
F = 64
BN = 400


def _sc_mesh():
    return plsc.VectorSubcoreMesh(core_axis_name="c", subcore_axis_name="s")


NDEG = 10240


def _deg_body(dstidx, degp, dst_v, ones_v, zbuf, acc):
    core = lax.axis_index("c")
    sub = lax.axis_index("s")
    wid = sub * NC + core
    pltpu.sync_copy(dstidx.at[wid], dst_v)

    def _fill(i, _):
        ones_v[pl.ds(i * 16, 16)] = jnp.full((16,), 1.0, jnp.float32)
        return _

    def _zero(i, _):
        zbuf[pl.ds(i * 16, 16)] = jnp.zeros((16,), jnp.float32)
        return _

    lax.fori_loop(0, CHUNK // 16, _fill, None)
    lax.fori_loop(0, (NDEG // NS) // 16, _zero, None)
    pltpu.sync_copy(zbuf, acc.at[pl.ds(sub * (NDEG // NS), NDEG // NS)])
    plsc.subcore_barrier()

    def _scat(j, _):
        pltpu.sync_copy(ones_v, acc.at[dst_v.at[j]], add=True)
        return _

    lax.fori_loop(0, NCH, _scat, None)
    plsc.subcore_barrier()
    for k in range(NC):
        @pl.when(core == k)
        def _(k=k):
            pltpu.sync_copy(acc.at[pl.ds(sub * (NDEG // NS), NDEG // NS)],
                            degp.at[k, pl.ds(sub * (NDEG // NS), NDEG // NS)])


def _deg_kernel(dstidx):
    return pl.kernel(
        _deg_body,
        out_type=jax.ShapeDtypeStruct((NC, NDEG), jnp.float32),
        mesh=_sc_mesh(),
        compiler_params=pltpu.CompilerParams(use_tc_tiling_on_sc=False),
        scratch_types=[
            pltpu.VMEM((NCH, CHUNK), jnp.int32),
            pltpu.VMEM((CHUNK,), jnp.float32),
            pltpu.VMEM((NDEG // NS,), jnp.float32),
            pltpu.VMEM_SHARED((NDEG,), jnp.float32),
        ],
    )(dstidx)


NBUF = 5


def _prop_body(C, g2, srcidx, dstidx, out, src_v, dst_v, bufs, zbuf, acc, sem):
    core = lax.axis_index("c")
    sub = lax.axis_index("s")
    wid = sub * NC + core
    pltpu.sync_copy(srcidx.at[wid], src_v)
    pltpu.sync_copy(dstidx.at[wid], dst_v)

    GPR = F // 16

    def _zb(i, _):
        zbuf[i // GPR, pl.ds((i % GPR) * 16, 16)] = jnp.zeros((16,), jnp.float32)
        return _

    lax.fori_loop(0, 160 * GPR, _zb, None)

    for cc in range(C):
        for q in range(5):
            pltpu.sync_copy(zbuf, acc.at[pl.ds(sub * STRIPE + q * 160, 160)])
        plsc.subcore_barrier()

        def _grp(t, _):
            j0 = t * NBUF
            gds = [pltpu.async_copy(g2.at[cc].at[src_v.at[j0 + b]],
                                    bufs.at[b], sem.at[b])
                   for b in range(NBUF)]
            sds = []
            for b in range(NBUF):
                gds[b].wait()
                sds.append(pltpu.async_copy(bufs.at[b],
                                            acc.at[dst_v.at[j0 + b]],
                                            sem.at[b], add=True))
            for sd in sds:
                sd.wait()
            return _

        lax.fori_loop(0, NCH // NBUF, _grp, None)
        plsc.subcore_barrier()
        for k in range(NC):
            @pl.when(core == k)
            def _(k=k, cc=cc):
                pltpu.sync_copy(
                    acc.at[pl.ds(sub * STRIPE, STRIPE)],
                    out.at[k, cc, pl.ds(sub * STRIPE, STRIPE)])
        plsc.subcore_barrier()


def _prop(C, g2, srcidx, dstidx):
    return pl.kernel(
        functools.partial(_prop_body, C),
        out_type=jax.ShapeDtypeStruct((NC, C, NPAD, F), jnp.float32),
        mesh=_sc_mesh(),
        compiler_params=pltpu.CompilerParams(use_tc_tiling_on_sc=False),
        scratch_types=[
            pltpu.VMEM((NCH, CHUNK), jnp.int32),
            pltpu.VMEM((NCH, CHUNK), jnp.int32),
            pltpu.VMEM((NBUF, CHUNK, F), jnp.float32),
            pltpu.VMEM((160, F), jnp.float32),
            pltpu.VMEM_SHARED((NPAD, F), jnp.float32),
            pltpu.SemaphoreType.DMA((NBUF,)),
        ],
    )(g2, srcidx, dstidx)


def _ta_body(degp_ref, x_ref, g_ref, norm_ref):
    deg = degp_ref[0] + degp_ref[1]
    nrm = lax.rsqrt(jnp.maximum(deg, 1.0))
    norm_ref[...] = nrm
    s = x_ref[...] * nrm
    for c in range(IN_FEATS // F):
        g_ref[c] = s[:, c * F:(c + 1) * F]


def _tc_prescale(degp, features):
    CI = IN_FEATS // F
    return pl.pallas_call(
        _ta_body,
        grid=(N // BN,),
        in_specs=[
            pl.BlockSpec((NC, BN, 1), lambda i: (0, i, 0)),
            pl.BlockSpec((BN, IN_FEATS), lambda i: (i, 0)),
        ],
        out_specs=[
            pl.BlockSpec((CI, BN, F), lambda i: (0, i, 0)),
            pl.BlockSpec((BN, 1), lambda i: (i, 0)),
        ],
        out_shape=[
            jax.ShapeDtypeStruct((CI, NPAD, F), jnp.float32),
            jax.ShapeDtypeStruct((N, 1), jnp.float32),
        ],
    )(degp[:, :N, None], features)


def _tb_body(p_ref, norm_ref, m_ref):
    nrm = norm_ref[...]
    m_ref[0] = (p_ref[0, 0] + p_ref[1, 0]) * (nrm * nrm)


def _tc_mid(p4, norm, C):
    return pl.pallas_call(
        _tb_body,
        grid=(C, N // BN),
        in_specs=[
            pl.BlockSpec((NC, 1, BN, F), lambda c, i: (0, c, i, 0)),
            pl.BlockSpec((BN, 1), lambda c, i: (i, 0)),
        ],
        out_specs=pl.BlockSpec((1, BN, F), lambda c, i: (c, i, 0)),
        out_shape=jax.ShapeDtypeStruct((C, NPAD, F), jnp.float32),
    )(p4, norm)


def _tc_layer_body(CI, CO, p_ref, norm_ref, w_ref, g_ref):
    nrm = norm_ref[...]
    acc = jnp.zeros((BN, w_ref.shape[0]), jnp.float32)
    for c in range(CI):
        t = (p_ref[0, c] + p_ref[1, c]) * nrm
        acc = acc + lax.dot_general(
            t, w_ref[:, c * F:(c + 1) * F],
            (((1,), (1,)), ((), ())), preferred_element_type=jnp.float32)
    h = jnp.maximum(acc, 0.0) * nrm
    for co in range(CO):
        g_ref[co] = h[:, co * F:(co + 1) * F]


def _tc_layer(p4, norm, W, CI, CO):
    return pl.pallas_call(
        functools.partial(_tc_layer_body, CI, CO),
        grid=(N // BN,),
        in_specs=[
            pl.BlockSpec((NC, CI, BN, F), lambda i: (0, 0, i, 0)),
            pl.BlockSpec((BN, 1), lambda i: (i, 0)),
            pl.BlockSpec(W.shape, lambda i: (0, 0)),
        ],
        out_specs=pl.BlockSpec((CO, BN, F), lambda i: (0, i, 0)),
        out_shape=jax.ShapeDtypeStruct((CO, NPAD, F), jnp.float32),
    )(p4, norm, W)


def _tc_layer2_body(CI, CO, p_ref, norm_ref, w2_ref, w3_ref, g_ref):
    nrm = norm_ref[...]
    acc = jnp.zeros((BN, N_HIDDEN), jnp.float32)
    for c in range(CI):
        t = (p_ref[0, c] + p_ref[1, c]) * nrm
        acc = acc + lax.dot_general(
            t, w2_ref[:, c * F:(c + 1) * F],
            (((1,), (1,)), ((), ())), preferred_element_type=jnp.float32)
    h = jnp.maximum(acc, 0.0)
    z = lax.dot_general(h, w3_ref[...], (((1,), (1,)), ((), ())),
                        preferred_element_type=jnp.float32)
    g = z * nrm
    for co in range(CO):
        g_ref[co] = g[:, co * F:(co + 1) * F]


def _tc_layer2(p4, norm, W2, W3, CI, CO):
    return pl.pallas_call(
        functools.partial(_tc_layer2_body, CI, CO),
        grid=(N // BN,),
        in_specs=[
            pl.BlockSpec((NC, CI, BN, F), lambda i: (0, 0, i, 0)),
            pl.BlockSpec((BN, 1), lambda i: (i, 0)),
            pl.BlockSpec(W2.shape, lambda i: (0, 0)),
            pl.BlockSpec(W3.shape, lambda i: (0, 0)),
        ],
        out_specs=pl.BlockSpec((CO, BN, F), lambda i: (0, i, 0)),
        out_shape=jax.ShapeDtypeStruct((CO, NPAD, F), jnp.float32),
    )(p4, norm, W2, W3)


def _td_body(p_ref, norm_ref, o_ref):
    nrm = norm_ref[...]
    cols = [(p_ref[0, c] + p_ref[1, c]) * nrm for c in range(N_CLASSES // F)]
    o_ref[...] = jnp.concatenate(cols, axis=1)


def _tc_final(p4, norm):
    CI = N_CLASSES // F
    return pl.pallas_call(
        _td_body,
        grid=(N // BN,),
        in_specs=[
            pl.BlockSpec((NC, CI, BN, F), lambda i: (0, 0, i, 0)),
            pl.BlockSpec((BN, 1), lambda i: (i, 0)),
        ],
        out_specs=pl.BlockSpec((BN, N_CLASSES), lambda i: (i, 0)),
        out_shape=jax.ShapeDtypeStruct((N, N_CLASSES), jnp.float32),
    )(p4, norm)


def kernel(features, edge_index, W1, W2, W3):
    src = edge_index[0]
    dst = edge_index[1]

    w = jnp.arange(NW, dtype=jnp.int32)[:, None]
    i = jnp.arange(PADE, dtype=jnp.int32)[None, :]
    pad_src = (w * 997 + i * 131) % N
    pad_dst = N + (i % 8) + jnp.zeros((NW, 1), jnp.int32)
    srcp = jnp.concatenate([src.reshape(NW, EPW), pad_src], axis=1)
    dstp = jnp.concatenate([dst.reshape(NW, EPW), pad_dst], axis=1)
    dsti = dstp.reshape(NW, NCH, CHUNK)

    srci = srcp.reshape(NW, NCH, CHUNK)

    degp = _deg_kernel(dsti)

    CA = IN_FEATS // F
    CB = N_HIDDEN // F
    g, norm = _tc_prescale(degp, features)
    p = _prop(CA, g, srci, dsti)
    m = _tc_mid(p, norm, CA)
    p = _prop(CA, m, srci, dsti)
    g = _tc_layer(p, norm, W1, CA, CB)
    p = _prop(CB, g, srci, dsti)
    m = _tc_mid(p, norm, CB)
    p = _prop(CB, m, srci, dsti)
    g = _tc_layer2(p, norm, W2, W3, CB, CA)
    p = _prop(CA, g, srci, dsti)
    m = _tc_mid(p, norm, CA)
    p = _prop(CA, m, srci, dsti)
    return _tc_final(p, norm)


# speedup vs baseline: 1.1928x; 1.1928x over previous
import functools

import jax
import jax.numpy as jnp
from jax import lax
from jax.experimental import pallas as pl
from jax.experimental.pallas import tpu as pltpu
from jax.experimental.pallas import tpu_sc as plsc

N = 10000
E = 160000
IN_FEATS = 256
N_HIDDEN = 512
N_CLASSES = 256

NC = 2
NS = 16
NW = NC * NS
EPW = E // NW
CHUNK = 128
NCH = -(-EPW // CHUNK)
EPW_PAD = NCH * CHUNK
PADE = EPW_PAD - EPW
NPAD = 10240
STRIPE = NPAD // NS
F = 64
BN = 1000


def _sc_mesh():
    return plsc.VectorSubcoreMesh(core_axis_name="c", subcore_axis_name="s")


NDEG = 10240


def _deg_body(dstidx, degp, dst_v, ones_v, zbuf, acc):
    core = lax.axis_index("c")
    sub = lax.axis_index("s")
    wid = sub * NC + core
    pltpu.sync_copy(dstidx.at[wid], dst_v)

    def _fill(i, _):
        ones_v[pl.ds(i * 16, 16)] = jnp.full((16,), 1.0, jnp.float32)
        return _

    def _zero(i, _):
        zbuf[pl.ds(i * 16, 16)] = jnp.zeros((16,), jnp.float32)
        return _

    lax.fori_loop(0, CHUNK // 16, _fill, None)
    lax.fori_loop(0, (NDEG // NS) // 16, _zero, None)
    pltpu.sync_copy(zbuf, acc.at[pl.ds(sub * (NDEG // NS), NDEG // NS)])
    plsc.subcore_barrier()

    def _scat(j, _):
        pltpu.sync_copy(ones_v, acc.at[dst_v.at[j]], add=True)
        return _

    lax.fori_loop(0, NCH, _scat, None)
    plsc.subcore_barrier()
    for k in range(NC):
        @pl.when(core == k)
        def _(k=k):
            pltpu.sync_copy(acc.at[pl.ds(sub * (NDEG // NS), NDEG // NS)],
                            degp.at[k, pl.ds(sub * (NDEG // NS), NDEG // NS)])


def _deg_kernel(dstidx):
    return pl.kernel(
        _deg_body,
        out_type=jax.ShapeDtypeStruct((NC, NDEG), jnp.float32),
        mesh=_sc_mesh(),
        compiler_params=pltpu.CompilerParams(use_tc_tiling_on_sc=False),
        scratch_types=[
            pltpu.VMEM((NCH, CHUNK), jnp.int32),
            pltpu.VMEM((CHUNK,), jnp.float32),
            pltpu.VMEM((NDEG // NS,), jnp.float32),
            pltpu.VMEM_SHARED((NDEG,), jnp.float32),
        ],
    )(dstidx)


NBUF = 8


def _prop_body(C, g2, srcidx, dstidx, out, src_v, dst_v, bufs, zbuf, acc, sem):
    core = lax.axis_index("c")
    sub = lax.axis_index("s")
    wid = sub * NC + core
    pltpu.sync_copy(srcidx.at[wid], src_v)
    pltpu.sync_copy(dstidx.at[wid], dst_v)

    GPR = F // 16

    def _zb(i, _):
        zbuf[i // GPR, pl.ds((i % GPR) * 16, 16)] = jnp.zeros((16,), jnp.float32)
        return _

    lax.fori_loop(0, 160 * GPR, _zb, None)

    for cc in range(C):
        for q in range(4):
            pltpu.sync_copy(zbuf, acc.at[pl.ds(sub * STRIPE + q * 160, 160)])
        plsc.subcore_barrier()

        def _grp(t, _):
            j0 = t * NBUF
            gds = [pltpu.async_copy(g2.at[cc].at[src_v.at[j0 + b]],
                                    bufs.at[b], sem.at[b])
                   for b in range(NBUF)]
            sds = []
            for b in range(NBUF):
                gds[b].wait()
                sds.append(pltpu.async_copy(bufs.at[b],
                                            acc.at[dst_v.at[j0 + b]],
                                            sem.at[b], add=True))
            for sd in sds:
                sd.wait()
            return _

        lax.fori_loop(0, NCH // NBUF, _grp, None)
        plsc.subcore_barrier()
        for k in range(NC):
            @pl.when(core == k)
            def _(k=k, cc=cc):
                pltpu.sync_copy(
                    acc.at[pl.ds(sub * STRIPE, STRIPE)],
                    out.at[k, cc, pl.ds(sub * STRIPE, STRIPE)])
        plsc.subcore_barrier()


def _prop(C, g2, srcidx, dstidx):
    return pl.kernel(
        functools.partial(_prop_body, C),
        out_type=jax.ShapeDtypeStruct((NC, C, NPAD, F), jnp.float32),
        mesh=_sc_mesh(),
        compiler_params=pltpu.CompilerParams(use_tc_tiling_on_sc=False),
        scratch_types=[
            pltpu.VMEM((NCH, CHUNK), jnp.int32),
            pltpu.VMEM((NCH, CHUNK), jnp.int32),
            pltpu.VMEM((NBUF, CHUNK, F), jnp.float32),
            pltpu.VMEM((160, F), jnp.float32),
            pltpu.VMEM_SHARED((NPAD, F), jnp.float32),
            pltpu.SemaphoreType.DMA((NBUF,)),
        ],
    )(g2, srcidx, dstidx)


def _ta_body(degp_ref, x_ref, g_ref, norm_ref):
    deg = degp_ref[0] + degp_ref[1]
    nrm = lax.rsqrt(jnp.maximum(deg, 1.0))
    norm_ref[...] = nrm
    s = x_ref[...] * nrm
    for c in range(IN_FEATS // F):
        g_ref[c] = s[:, c * F:(c + 1) * F]


def _tc_prescale(degp, features):
    CI = IN_FEATS // F
    return pl.pallas_call(
        _ta_body,
        grid=(N // BN,),
        in_specs=[
            pl.BlockSpec((NC, BN, 1), lambda i: (0, i, 0)),
            pl.BlockSpec((BN, IN_FEATS), lambda i: (i, 0)),
        ],
        out_specs=[
            pl.BlockSpec((CI, BN, F), lambda i: (0, i, 0)),
            pl.BlockSpec((BN, 1), lambda i: (i, 0)),
        ],
        out_shape=[
            jax.ShapeDtypeStruct((CI, NPAD, F), jnp.float32),
            jax.ShapeDtypeStruct((N, 1), jnp.float32),
        ],
    )(degp[:, :N, None], features)


def _tb_body(p_ref, norm_ref, m_ref):
    nrm = norm_ref[...]
    m_ref[0] = (p_ref[0, 0] + p_ref[1, 0]) * (nrm * nrm)


def _tc_mid(p4, norm, C):
    return pl.pallas_call(
        _tb_body,
        grid=(C, N // BN),
        in_specs=[
            pl.BlockSpec((NC, 1, BN, F), lambda c, i: (0, c, i, 0)),
            pl.BlockSpec((BN, 1), lambda c, i: (i, 0)),
        ],
        out_specs=pl.BlockSpec((1, BN, F), lambda c, i: (c, i, 0)),
        out_shape=jax.ShapeDtypeStruct((C, NPAD, F), jnp.float32),
    )(p4, norm)


def _tc_layer_body(CI, CO, p_ref, norm_ref, w_ref, g_ref):
    nrm = norm_ref[...]
    acc = jnp.zeros((BN, w_ref.shape[0]), jnp.float32)
    for c in range(CI):
        t = (p_ref[0, c] + p_ref[1, c]) * nrm
        acc = acc + lax.dot_general(
            t, w_ref[:, c * F:(c + 1) * F],
            (((1,), (1,)), ((), ())), preferred_element_type=jnp.float32)
    h = jnp.maximum(acc, 0.0) * nrm
    for co in range(CO):
        g_ref[co] = h[:, co * F:(co + 1) * F]


def _tc_layer(p4, norm, W, CI, CO):
    return pl.pallas_call(
        functools.partial(_tc_layer_body, CI, CO),
        grid=(N // BN,),
        in_specs=[
            pl.BlockSpec((NC, CI, BN, F), lambda i: (0, 0, i, 0)),
            pl.BlockSpec((BN, 1), lambda i: (i, 0)),
            pl.BlockSpec(W.shape, lambda i: (0, 0)),
        ],
        out_specs=pl.BlockSpec((CO, BN, F), lambda i: (0, i, 0)),
        out_shape=jax.ShapeDtypeStruct((CO, NPAD, F), jnp.float32),
    )(p4, norm, W)


def _tc_layer2_body(CI, CO, p_ref, norm_ref, w2_ref, w3_ref, g_ref):
    nrm = norm_ref[...]
    acc = jnp.zeros((BN, N_HIDDEN), jnp.float32)
    for c in range(CI):
        t = (p_ref[0, c] + p_ref[1, c]) * nrm
        acc = acc + lax.dot_general(
            t, w2_ref[:, c * F:(c + 1) * F],
            (((1,), (1,)), ((), ())), preferred_element_type=jnp.float32)
    h = jnp.maximum(acc, 0.0)
    z = lax.dot_general(h, w3_ref[...], (((1,), (1,)), ((), ())),
                        preferred_element_type=jnp.float32)
    g = z * nrm
    for co in range(CO):
        g_ref[co] = g[:, co * F:(co + 1) * F]


def _tc_layer2(p4, norm, W2, W3, CI, CO):
    return pl.pallas_call(
        functools.partial(_tc_layer2_body, CI, CO),
        grid=(N // BN,),
        in_specs=[
            pl.BlockSpec((NC, CI, BN, F), lambda i: (0, 0, i, 0)),
            pl.BlockSpec((BN, 1), lambda i: (i, 0)),
            pl.BlockSpec(W2.shape, lambda i: (0, 0)),
            pl.BlockSpec(W3.shape, lambda i: (0, 0)),
        ],
        out_specs=pl.BlockSpec((CO, BN, F), lambda i: (0, i, 0)),
        out_shape=jax.ShapeDtypeStruct((CO, NPAD, F), jnp.float32),
    )(p4, norm, W2, W3)


def _td_body(p_ref, norm_ref, o_ref):
    nrm = norm_ref[...]
    cols = [(p_ref[0, c] + p_ref[1, c]) * nrm for c in range(N_CLASSES // F)]
    o_ref[...] = jnp.concatenate(cols, axis=1)


def _tc_final(p4, norm):
    CI = N_CLASSES // F
    return pl.pallas_call(
        _td_body,
        grid=(N // BN,),
        in_specs=[
            pl.BlockSpec((NC, CI, BN, F), lambda i: (0, 0, i, 0)),
            pl.BlockSpec((BN, 1), lambda i: (i, 0)),
        ],
        out_specs=pl.BlockSpec((BN, N_CLASSES), lambda i: (i, 0)),
        out_shape=jax.ShapeDtypeStruct((N, N_CLASSES), jnp.float32),
    )(p4, norm)


def kernel(features, edge_index, W1, W2, W3):
    src = edge_index[0]
    dst = edge_index[1]

    w = jnp.arange(NW, dtype=jnp.int32)[:, None]
    i = jnp.arange(PADE, dtype=jnp.int32)[None, :]
    pad_src = (w * 997 + i * 131) % N
    pad_dst = N + (i % 8) + jnp.zeros((NW, 1), jnp.int32)
    srcp = jnp.concatenate([src.reshape(NW, EPW), pad_src], axis=1)
    dstp = jnp.concatenate([dst.reshape(NW, EPW), pad_dst], axis=1)
    dsti = dstp.reshape(NW, NCH, CHUNK)

    srci = srcp.reshape(NW, NCH, CHUNK)

    degp = _deg_kernel(dsti)

    CA = IN_FEATS // F
    CB = N_HIDDEN // F
    g, norm = _tc_prescale(degp, features)
    p = _prop(CA, g, srci, dsti)
    m = _tc_mid(p, norm, CA)
    p = _prop(CA, m, srci, dsti)
    g = _tc_layer(p, norm, W1, CA, CB)
    p = _prop(CB, g, srci, dsti)
    m = _tc_mid(p, norm, CB)
    p = _prop(CB, m, srci, dsti)
    g = _tc_layer2(p, norm, W2, W3, CB, CA)
    p = _prop(CA, g, srci, dsti)
    m = _tc_mid(p, norm, CA)
    p = _prop(CA, m, srci, dsti)
    return _tc_final(p, norm)


# speedup vs baseline: 1.5634x; 1.3107x over previous
import functools

import jax
import jax.numpy as jnp
from jax import lax
from jax.experimental import pallas as pl
from jax.experimental.pallas import tpu as pltpu
from jax.experimental.pallas import tpu_sc as plsc

N = 10000
E = 160000
IN_FEATS = 256
N_HIDDEN = 512
N_CLASSES = 256

NC = 2
NS = 16
NW = NC * NS
EPW = E // NW
CHUNK = 128
NCH = -(-EPW // CHUNK)
EPW_PAD = NCH * CHUNK
PADE = EPW_PAD - EPW
NPAD = 10240
HALF = NPAD // 2
STRIPE = NPAD // NS
F = 64
BP = 512
NBUF = 8


def _sc_mesh():
    return plsc.VectorSubcoreMesh(core_axis_name="c", subcore_axis_name="s")


def _deg_body(dstidx, degp, dst_v, ones_v, zbuf, acc):
    core = lax.axis_index("c")
    sub = lax.axis_index("s")
    wid = sub * NC + core
    pltpu.sync_copy(dstidx.at[wid], dst_v)

    def _fill(i, _):
        ones_v[pl.ds(i * 16, 16)] = jnp.full((16,), 1.0, jnp.float32)
        return _

    def _zero(i, _):
        zbuf[pl.ds(i * 16, 16)] = jnp.zeros((16,), jnp.float32)
        return _

    lax.fori_loop(0, CHUNK // 16, _fill, None)
    lax.fori_loop(0, STRIPE // 16, _zero, None)
    pltpu.sync_copy(zbuf, acc.at[pl.ds(sub * STRIPE, STRIPE)])
    plsc.subcore_barrier()

    def _scat(j, _):
        pltpu.sync_copy(ones_v, acc.at[dst_v.at[j]], add=True)
        return _

    lax.fori_loop(0, NCH, _scat, None)
    plsc.subcore_barrier()
    for k in range(NC):
        @pl.when(core == k)
        def _(k=k):
            pltpu.sync_copy(acc.at[pl.ds(sub * STRIPE, STRIPE)],
                            degp.at[k, pl.ds(sub * STRIPE, STRIPE)])


def _deg_kernel(dstidx):
    return pl.kernel(
        _deg_body,
        out_type=jax.ShapeDtypeStruct((NC, NPAD), jnp.float32),
        mesh=_sc_mesh(),
        compiler_params=pltpu.CompilerParams(use_tc_tiling_on_sc=False),
        scratch_types=[
            pltpu.VMEM((NCH, CHUNK), jnp.int32),
            pltpu.VMEM((CHUNK,), jnp.float32),
            pltpu.VMEM((STRIPE,), jnp.float32),
            pltpu.VMEM_SHARED((NPAD,), jnp.float32),
        ],
    )(dstidx)


def _prop_body(C, g2, srcidx, dstidx, out, src_v, dst_v, bufs, zbuf, acc, sem):
    core = lax.axis_index("c")
    sub = lax.axis_index("s")
    wid = sub * NC + core
    pltpu.sync_copy(srcidx.at[wid], src_v)
    pltpu.sync_copy(dstidx.at[wid], dst_v)

    GPR = F // 16

    def _zb(i, _):
        zbuf[i // GPR, pl.ds((i % GPR) * 16, 16)] = jnp.zeros((16,), jnp.float32)
        return _

    lax.fori_loop(0, 160 * GPR, _zb, None)

    for cc in range(C):
        for q in range(4):
            pltpu.sync_copy(zbuf, acc.at[pl.ds(sub * STRIPE + q * 160, 160)])
        plsc.subcore_barrier()

        def _grp(t, _):
            j0 = t * NBUF
            gds = [pltpu.async_copy(g2.at[cc].at[src_v.at[j0 + b]],
                                    bufs.at[b], sem.at[b])
                   for b in range(NBUF)]
            sds = []
            for b in range(NBUF):
                gds[b].wait()
                sds.append(pltpu.async_copy(bufs.at[b],
                                            acc.at[dst_v.at[j0 + b]],
                                            sem.at[b], add=True))
            for sd in sds:
                sd.wait()
            return _

        lax.fori_loop(0, NCH // NBUF, _grp, None)
        plsc.subcore_barrier()
        for k in range(NC):
            @pl.when(core == k)
            def _(k=k, cc=cc):
                pltpu.sync_copy(
                    acc.at[pl.ds(sub * STRIPE, STRIPE)],
                    out.at[k, cc, pl.ds(sub * STRIPE, STRIPE)])
        plsc.subcore_barrier()


def _prop(C, g2, srcidx, dstidx):
    return pl.kernel(
        functools.partial(_prop_body, C),
        out_type=jax.ShapeDtypeStruct((NC, C, NPAD, F), jnp.float32),
        mesh=_sc_mesh(),
        compiler_params=pltpu.CompilerParams(use_tc_tiling_on_sc=False),
        scratch_types=[
            pltpu.VMEM((NCH, CHUNK), jnp.int32),
            pltpu.VMEM((NCH, CHUNK), jnp.int32),
            pltpu.VMEM((NBUF, CHUNK, F), jnp.float32),
            pltpu.VMEM((160, F), jnp.float32),
            pltpu.VMEM_SHARED((NPAD, F), jnp.float32),
            pltpu.SemaphoreType.DMA((NBUF,)),
        ],
    )(g2, srcidx, dstidx)


def _ta_body(degp_ref, xt_ref, xb_ref, gp_ref, normw_ref):
    deg = degp_ref[0] + degp_ref[1]
    nrm = lax.rsqrt(jnp.maximum(deg, 1.0))
    nt, nb = nrm[:, 0:1], nrm[:, 1:2]
    normw = jnp.concatenate(
        [jnp.broadcast_to(nt, (BP, F)), jnp.broadcast_to(nb, (BP, F))], axis=1)
    normw_ref[...] = normw
    st = xt_ref[...] * nt
    sb = xb_ref[...] * nb
    for c in range(IN_FEATS // F):
        gp_ref[c] = jnp.concatenate(
            [st[:, c * F:(c + 1) * F], sb[:, c * F:(c + 1) * F]], axis=1)


def _tc_prescale(degs, featpad):
    CI = IN_FEATS // F
    return pl.pallas_call(
        _ta_body,
        grid=(HALF // BP,),
        in_specs=[
            pl.BlockSpec((NC, BP, 2), lambda i: (0, i, 0)),
            pl.BlockSpec((BP, IN_FEATS), lambda i: (i, 0)),
            pl.BlockSpec((BP, IN_FEATS), lambda i: (HALF // BP + i, 0)),
        ],
        out_specs=[
            pl.BlockSpec((CI, BP, 2 * F), lambda i: (0, i, 0)),
            pl.BlockSpec((BP, 2 * F), lambda i: (i, 0)),
        ],
        out_shape=[
            jax.ShapeDtypeStruct((CI, HALF, 2 * F), jnp.float32),
            jax.ShapeDtypeStruct((HALF, 2 * F), jnp.float32),
        ],
    )(degs, featpad, featpad)


def _tb_body(pp_ref, normw_ref, m_ref):
    nw = normw_ref[...]
    m_ref[0] = (pp_ref[0, 0] + pp_ref[1, 0]) * (nw * nw)


def _tc_mid(pp, normw, C):
    return pl.pallas_call(
        _tb_body,
        grid=(C, HALF // BP),
        in_specs=[
            pl.BlockSpec((NC, 1, BP, 2 * F), lambda c, i: (0, c, i, 0)),
            pl.BlockSpec((BP, 2 * F), lambda c, i: (i, 0)),
        ],
        out_specs=pl.BlockSpec((1, BP, 2 * F), lambda c, i: (c, i, 0)),
        out_shape=jax.ShapeDtypeStruct((C, HALF, 2 * F), jnp.float32),
    )(pp, normw)


def _dotT(x, w):
    return lax.dot_general(x, w, (((1,), (1,)), ((), ())),
                           preferred_element_type=jnp.float32)


def _tc_layer_body(CI, CO, pp_ref, normw_ref, w_ref, gp_ref):
    nw = normw_ref[...]
    dout = w_ref.shape[0]
    acct = jnp.zeros((BP, dout), jnp.float32)
    accb = jnp.zeros((BP, dout), jnp.float32)
    for c in range(CI):
        t = (pp_ref[0, c] + pp_ref[1, c]) * nw
        wc = w_ref[:, c * F:(c + 1) * F]
        acct = acct + _dotT(t[:, :F], wc)
        accb = accb + _dotT(t[:, F:], wc)
    ht = jnp.maximum(acct, 0.0) * nw[:, 0:1]
    hb = jnp.maximum(accb, 0.0) * nw[:, F:F + 1]
    for co in range(CO):
        gp_ref[co] = jnp.concatenate(
            [ht[:, co * F:(co + 1) * F], hb[:, co * F:(co + 1) * F]], axis=1)


def _tc_layer(pp, normw, W, CI, CO):
    return pl.pallas_call(
        functools.partial(_tc_layer_body, CI, CO),
        grid=(HALF // BP,),
        in_specs=[
            pl.BlockSpec((NC, CI, BP, 2 * F), lambda i: (0, 0, i, 0)),
            pl.BlockSpec((BP, 2 * F), lambda i: (i, 0)),
            pl.BlockSpec(W.shape, lambda i: (0, 0)),
        ],
        out_specs=pl.BlockSpec((CO, BP, 2 * F), lambda i: (0, i, 0)),
        out_shape=jax.ShapeDtypeStruct((CO, HALF, 2 * F), jnp.float32),
    )(pp, normw, W)


def _tc_layer2_body(CI, CO, pp_ref, normw_ref, w2_ref, w3_ref, gp_ref):
    nw = normw_ref[...]
    acct = jnp.zeros((BP, N_HIDDEN), jnp.float32)
    accb = jnp.zeros((BP, N_HIDDEN), jnp.float32)
    for c in range(CI):
        t = (pp_ref[0, c] + pp_ref[1, c]) * nw
        wc = w2_ref[:, c * F:(c + 1) * F]
        acct = acct + _dotT(t[:, :F], wc)
        accb = accb + _dotT(t[:, F:], wc)
    zt = _dotT(jnp.maximum(acct, 0.0), w3_ref[...]) * nw[:, 0:1]
    zb = _dotT(jnp.maximum(accb, 0.0), w3_ref[...]) * nw[:, F:F + 1]
    for co in range(CO):
        gp_ref[co] = jnp.concatenate(
            [zt[:, co * F:(co + 1) * F], zb[:, co * F:(co + 1) * F]], axis=1)


def _tc_layer2(pp, normw, W2, W3, CI, CO):
    return pl.pallas_call(
        functools.partial(_tc_layer2_body, CI, CO),
        grid=(HALF // BP,),
        in_specs=[
            pl.BlockSpec((NC, CI, BP, 2 * F), lambda i: (0, 0, i, 0)),
            pl.BlockSpec((BP, 2 * F), lambda i: (i, 0)),
            pl.BlockSpec(W2.shape, lambda i: (0, 0)),
            pl.BlockSpec(W3.shape, lambda i: (0, 0)),
        ],
        out_specs=pl.BlockSpec((CO, BP, 2 * F), lambda i: (0, i, 0)),
        out_shape=jax.ShapeDtypeStruct((CO, HALF, 2 * F), jnp.float32),
    )(pp, normw, W2, W3)


def _td_body(pp_ref, normw_ref, ot_ref, ob_ref):
    nw = normw_ref[...]
    ts, bs = [], []
    for c in range(N_CLASSES // F):
        t = (pp_ref[0, c] + pp_ref[1, c]) * nw
        ts.append(t[:, :F])
        bs.append(t[:, F:])
    ot_ref[...] = jnp.concatenate(ts, axis=1)
    ob_ref[...] = jnp.concatenate(bs, axis=1)


def _tc_final(pp, normw):
    CI = N_CLASSES // F
    return pl.pallas_call(
        _td_body,
        grid=(HALF // BP,),
        in_specs=[
            pl.BlockSpec((NC, CI, BP, 2 * F), lambda i: (0, 0, i, 0)),
            pl.BlockSpec((BP, 2 * F), lambda i: (i, 0)),
        ],
        out_specs=[
            pl.BlockSpec((BP, N_CLASSES), lambda i: (i, 0)),
            pl.BlockSpec((BP, N_CLASSES), lambda i: (i, 0)),
        ],
        out_shape=[
            jax.ShapeDtypeStruct((HALF, N_CLASSES), jnp.float32),
            jax.ShapeDtypeStruct((HALF, N_CLASSES), jnp.float32),
        ],
    )(pp, normw)


def kernel(features, edge_index, W1, W2, W3):
    src = edge_index[0]
    dst = edge_index[1]

    def sig(v):
        return jnp.where(v < HALF, 2 * v, 2 * v - (2 * HALF - 1))

    w = jnp.arange(NW, dtype=jnp.int32)[:, None]
    i = jnp.arange(PADE, dtype=jnp.int32)[None, :]
    pad_src = (w * 997 + i * 131) % N
    pad_dst = (NPAD - 1 - 2 * (i % 8)) + jnp.zeros((NW, 1), jnp.int32)
    srcp = jnp.concatenate([sig(src).reshape(NW, EPW), pad_src], axis=1)
    dstp = jnp.concatenate([sig(dst).reshape(NW, EPW), pad_dst], axis=1)
    srci = srcp.reshape(NW, NCH, CHUNK)
    dsti = dstp.reshape(NW, NCH, CHUNK)

    degp = _deg_kernel(dsti)
    degs = degp.reshape(NC, HALF, 2)
    featpad = jnp.pad(features, ((0, NPAD - N), (0, 0)))

    CA = IN_FEATS // F
    CB = N_HIDDEN // F

    def pair(p):
        return p.reshape(NC, p.shape[1], HALF, 2 * F)

    def flat(gp):
        return gp.reshape(gp.shape[0], NPAD, F)

    gp, normw = _tc_prescale(degs, featpad)
    p = _prop(CA, flat(gp), srci, dsti)
    m = _tc_mid(pair(p), normw, CA)
    p = _prop(CA, flat(m), srci, dsti)
    gp = _tc_layer(pair(p), normw, W1, CA, CB)
    p = _prop(CB, flat(gp), srci, dsti)
    m = _tc_mid(pair(p), normw, CB)
    p = _prop(CB, flat(m), srci, dsti)
    gp = _tc_layer2(pair(p), normw, W2, W3, CB, CA)
    p = _prop(CA, flat(gp), srci, dsti)
    m = _tc_mid(pair(p), normw, CA)
    p = _prop(CA, flat(m), srci, dsti)
    ot, ob = _tc_final(pair(p), normw)
    return jnp.concatenate([ot, ob[:N - HALF]], axis=0)


# speedup vs baseline: 1.5651x; 1.0011x over previous
import functools

import jax
import jax.numpy as jnp
from jax import lax
from jax.experimental import pallas as pl
from jax.experimental.pallas import tpu as pltpu
from jax.experimental.pallas import tpu_sc as plsc

N = 10000
E = 160000
IN_FEATS = 256
N_HIDDEN = 512
N_CLASSES = 256

NC = 2
NS = 16
NW = NC * NS
EPW = E // NW
CHUNK = 128
NCH = -(-EPW // CHUNK)
EPW_PAD = NCH * CHUNK
PADE = EPW_PAD - EPW
NPAD = 10240
HALF = NPAD // 2
STRIPE = NPAD // NS
F = 64
BP = 512
NBUF = 8


def _sc_mesh():
    return plsc.VectorSubcoreMesh(core_axis_name="c", subcore_axis_name="s")


def _deg_body(dstidx, degp, dst_v, ones_v, zbuf, acc):
    core = lax.axis_index("c")
    sub = lax.axis_index("s")
    wid = sub * NC + core
    pltpu.sync_copy(dstidx.at[wid], dst_v)

    def _fill(i, _):
        ones_v[pl.ds(i * 16, 16)] = jnp.full((16,), 1.0, jnp.float32)
        return _

    def _zero(i, _):
        zbuf[pl.ds(i * 16, 16)] = jnp.zeros((16,), jnp.float32)
        return _

    lax.fori_loop(0, CHUNK // 16, _fill, None)
    lax.fori_loop(0, STRIPE // 16, _zero, None)
    pltpu.sync_copy(zbuf, acc.at[pl.ds(sub * STRIPE, STRIPE)])
    plsc.subcore_barrier()

    def _scat(j, _):
        pltpu.sync_copy(ones_v, acc.at[dst_v.at[j]], add=True)
        return _

    lax.fori_loop(0, NCH, _scat, None)
    plsc.subcore_barrier()
    for k in range(NC):
        @pl.when(core == k)
        def _(k=k):
            pltpu.sync_copy(acc.at[pl.ds(sub * STRIPE, STRIPE)],
                            degp.at[k, pl.ds(sub * STRIPE, STRIPE)])


def _deg_kernel(dstidx):
    return pl.kernel(
        _deg_body,
        out_type=jax.ShapeDtypeStruct((NC, NPAD), jnp.float32),
        mesh=_sc_mesh(),
        compiler_params=pltpu.CompilerParams(use_tc_tiling_on_sc=False),
        scratch_types=[
            pltpu.VMEM((NCH, CHUNK), jnp.int32),
            pltpu.VMEM((CHUNK,), jnp.float32),
            pltpu.VMEM((STRIPE,), jnp.float32),
            pltpu.VMEM_SHARED((NPAD,), jnp.float32),
        ],
    )(dstidx)


def _prop_body(C, g2, srcidx, dstidx, out, src_v, dst_v, bufs, zbuf, acc, sem):
    core = lax.axis_index("c")
    sub = lax.axis_index("s")
    wid = sub * NC + core
    pltpu.sync_copy(srcidx.at[wid], src_v)
    pltpu.sync_copy(dstidx.at[wid], dst_v)

    GPR = F // 16

    def _zb(i, _):
        zbuf[i // GPR, pl.ds((i % GPR) * 16, 16)] = jnp.zeros((16,), jnp.float32)
        return _

    lax.fori_loop(0, 160 * GPR, _zb, None)

    for q in range(4):
        pltpu.sync_copy(zbuf, acc.at[pl.ds(sub * STRIPE + q * 160, 160)])
    plsc.subcore_barrier()

    for cc in range(C):
        def _grp(t, _):
            j0 = t * NBUF
            gds = [pltpu.async_copy(g2.at[cc].at[src_v.at[j0 + b]],
                                    bufs.at[b], sem.at[b])
                   for b in range(NBUF)]
            sds = []
            for b in range(NBUF):
                gds[b].wait()
                sds.append(pltpu.async_copy(bufs.at[b],
                                            acc.at[dst_v.at[j0 + b]],
                                            sem.at[b], add=True))
            for sd in sds:
                sd.wait()
            return _

        lax.fori_loop(0, NCH // NBUF, _grp, None)
        plsc.subcore_barrier()
        for k in range(NC):
            @pl.when(core == k)
            def _(k=k, cc=cc):
                pltpu.sync_copy(
                    acc.at[pl.ds(sub * STRIPE, STRIPE)],
                    out.at[k, cc, pl.ds(sub * STRIPE, STRIPE)])
        if cc + 1 < C:
            for q in range(4):
                pltpu.sync_copy(zbuf, acc.at[pl.ds(sub * STRIPE + q * 160, 160)])
        plsc.subcore_barrier()


def _prop(C, g2, srcidx, dstidx):
    return pl.kernel(
        functools.partial(_prop_body, C),
        out_type=jax.ShapeDtypeStruct((NC, C, NPAD, F), jnp.float32),
        mesh=_sc_mesh(),
        compiler_params=pltpu.CompilerParams(use_tc_tiling_on_sc=False),
        scratch_types=[
            pltpu.VMEM((NCH, CHUNK), jnp.int32),
            pltpu.VMEM((NCH, CHUNK), jnp.int32),
            pltpu.VMEM((NBUF, CHUNK, F), jnp.float32),
            pltpu.VMEM((160, F), jnp.float32),
            pltpu.VMEM_SHARED((NPAD, F), jnp.float32),
            pltpu.SemaphoreType.DMA((NBUF,)),
        ],
    )(g2, srcidx, dstidx)


def _ta_body(degp_ref, xt_ref, xb_ref, gp_ref, normw_ref):
    deg = degp_ref[0] + degp_ref[1]
    nrm = lax.rsqrt(jnp.maximum(deg, 1.0))
    nt, nb = nrm[:, 0:1], nrm[:, 1:2]
    normw = jnp.concatenate(
        [jnp.broadcast_to(nt, (BP, F)), jnp.broadcast_to(nb, (BP, F))], axis=1)
    normw_ref[...] = normw
    st = xt_ref[...] * nt
    sb = xb_ref[...] * nb
    for c in range(IN_FEATS // F):
        gp_ref[c] = jnp.concatenate(
            [st[:, c * F:(c + 1) * F], sb[:, c * F:(c + 1) * F]], axis=1)


def _tc_prescale(degs, featpad):
    CI = IN_FEATS // F
    return pl.pallas_call(
        _ta_body,
        grid=(HALF // BP,),
        in_specs=[
            pl.BlockSpec((NC, BP, 2), lambda i: (0, i, 0)),
            pl.BlockSpec((BP, IN_FEATS), lambda i: (i, 0)),
            pl.BlockSpec((BP, IN_FEATS), lambda i: (HALF // BP + i, 0)),
        ],
        out_specs=[
            pl.BlockSpec((CI, BP, 2 * F), lambda i: (0, i, 0)),
            pl.BlockSpec((BP, 2 * F), lambda i: (i, 0)),
        ],
        out_shape=[
            jax.ShapeDtypeStruct((CI, HALF, 2 * F), jnp.float32),
            jax.ShapeDtypeStruct((HALF, 2 * F), jnp.float32),
        ],
    )(degs, featpad, featpad)


def _tb_body(pp_ref, normw_ref, m_ref):
    nw = normw_ref[...]
    m_ref[0] = (pp_ref[0, 0] + pp_ref[1, 0]) * (nw * nw)


def _tc_mid(pp, normw, C):
    return pl.pallas_call(
        _tb_body,
        grid=(C, HALF // BP),
        in_specs=[
            pl.BlockSpec((NC, 1, BP, 2 * F), lambda c, i: (0, c, i, 0)),
            pl.BlockSpec((BP, 2 * F), lambda c, i: (i, 0)),
        ],
        out_specs=pl.BlockSpec((1, BP, 2 * F), lambda c, i: (c, i, 0)),
        out_shape=jax.ShapeDtypeStruct((C, HALF, 2 * F), jnp.float32),
    )(pp, normw)


def _dotT(x, w):
    return lax.dot_general(x, w, (((1,), (1,)), ((), ())),
                           preferred_element_type=jnp.float32)


def _tc_layer_body(CI, CO, pp_ref, normw_ref, w_ref, gp_ref):
    nw = normw_ref[...]
    dout = w_ref.shape[0]
    acct = jnp.zeros((BP, dout), jnp.float32)
    accb = jnp.zeros((BP, dout), jnp.float32)
    for c in range(CI):
        t = (pp_ref[0, c] + pp_ref[1, c]) * nw
        wc = w_ref[:, c * F:(c + 1) * F]
        acct = acct + _dotT(t[:, :F], wc)
        accb = accb + _dotT(t[:, F:], wc)
    ht = jnp.maximum(acct, 0.0) * nw[:, 0:1]
    hb = jnp.maximum(accb, 0.0) * nw[:, F:F + 1]
    for co in range(CO):
        gp_ref[co] = jnp.concatenate(
            [ht[:, co * F:(co + 1) * F], hb[:, co * F:(co + 1) * F]], axis=1)


def _tc_layer(pp, normw, W, CI, CO):
    return pl.pallas_call(
        functools.partial(_tc_layer_body, CI, CO),
        grid=(HALF // BP,),
        in_specs=[
            pl.BlockSpec((NC, CI, BP, 2 * F), lambda i: (0, 0, i, 0)),
            pl.BlockSpec((BP, 2 * F), lambda i: (i, 0)),
            pl.BlockSpec(W.shape, lambda i: (0, 0)),
        ],
        out_specs=pl.BlockSpec((CO, BP, 2 * F), lambda i: (0, i, 0)),
        out_shape=jax.ShapeDtypeStruct((CO, HALF, 2 * F), jnp.float32),
    )(pp, normw, W)


def _tc_layer2_body(CI, CO, pp_ref, normw_ref, w2_ref, w3_ref, gp_ref):
    nw = normw_ref[...]
    acct = jnp.zeros((BP, N_HIDDEN), jnp.float32)
    accb = jnp.zeros((BP, N_HIDDEN), jnp.float32)
    for c in range(CI):
        t = (pp_ref[0, c] + pp_ref[1, c]) * nw
        wc = w2_ref[:, c * F:(c + 1) * F]
        acct = acct + _dotT(t[:, :F], wc)
        accb = accb + _dotT(t[:, F:], wc)
    zt = _dotT(jnp.maximum(acct, 0.0), w3_ref[...]) * nw[:, 0:1]
    zb = _dotT(jnp.maximum(accb, 0.0), w3_ref[...]) * nw[:, F:F + 1]
    for co in range(CO):
        gp_ref[co] = jnp.concatenate(
            [zt[:, co * F:(co + 1) * F], zb[:, co * F:(co + 1) * F]], axis=1)


def _tc_layer2(pp, normw, W2, W3, CI, CO):
    return pl.pallas_call(
        functools.partial(_tc_layer2_body, CI, CO),
        grid=(HALF // BP,),
        in_specs=[
            pl.BlockSpec((NC, CI, BP, 2 * F), lambda i: (0, 0, i, 0)),
            pl.BlockSpec((BP, 2 * F), lambda i: (i, 0)),
            pl.BlockSpec(W2.shape, lambda i: (0, 0)),
            pl.BlockSpec(W3.shape, lambda i: (0, 0)),
        ],
        out_specs=pl.BlockSpec((CO, BP, 2 * F), lambda i: (0, i, 0)),
        out_shape=jax.ShapeDtypeStruct((CO, HALF, 2 * F), jnp.float32),
    )(pp, normw, W2, W3)


def _td_body(pp_ref, normw_ref, ot_ref, ob_ref):
    nw = normw_ref[...]
    ts, bs = [], []
    for c in range(N_CLASSES // F):
        t = (pp_ref[0, c] + pp_ref[1, c]) * nw
        ts.append(t[:, :F])
        bs.append(t[:, F:])
    ot_ref[...] = jnp.concatenate(ts, axis=1)
    ob_ref[...] = jnp.concatenate(bs, axis=1)


def _tc_final(pp, normw):
    CI = N_CLASSES // F
    return pl.pallas_call(
        _td_body,
        grid=(HALF // BP,),
        in_specs=[
            pl.BlockSpec((NC, CI, BP, 2 * F), lambda i: (0, 0, i, 0)),
            pl.BlockSpec((BP, 2 * F), lambda i: (i, 0)),
        ],
        out_specs=[
            pl.BlockSpec((BP, N_CLASSES), lambda i: (i, 0)),
            pl.BlockSpec((BP, N_CLASSES), lambda i: (i, 0)),
        ],
        out_shape=[
            jax.ShapeDtypeStruct((HALF, N_CLASSES), jnp.float32),
            jax.ShapeDtypeStruct((HALF, N_CLASSES), jnp.float32),
        ],
    )(pp, normw)


def kernel(features, edge_index, W1, W2, W3):
    src = edge_index[0]
    dst = edge_index[1]

    def sig(v):
        return jnp.where(v < HALF, 2 * v, 2 * v - (2 * HALF - 1))

    w = jnp.arange(NW, dtype=jnp.int32)[:, None]
    i = jnp.arange(PADE, dtype=jnp.int32)[None, :]
    pad_src = (w * 997 + i * 131) % N
    pad_dst = (NPAD - 1 - 2 * (i % 8)) + jnp.zeros((NW, 1), jnp.int32)
    srcp = jnp.concatenate([sig(src).reshape(NW, EPW), pad_src], axis=1)
    dstp = jnp.concatenate([sig(dst).reshape(NW, EPW), pad_dst], axis=1)
    srci = srcp.reshape(NW, NCH, CHUNK)
    dsti = dstp.reshape(NW, NCH, CHUNK)

    degp = _deg_kernel(dsti)
    degs = degp.reshape(NC, HALF, 2)
    featpad = jnp.pad(features, ((0, NPAD - N), (0, 0)))

    CA = IN_FEATS // F
    CB = N_HIDDEN // F

    def pair(p):
        return p.reshape(NC, p.shape[1], HALF, 2 * F)

    def flat(gp):
        return gp.reshape(gp.shape[0], NPAD, F)

    gp, normw = _tc_prescale(degs, featpad)
    p = _prop(CA, flat(gp), srci, dsti)
    m = _tc_mid(pair(p), normw, CA)
    p = _prop(CA, flat(m), srci, dsti)
    gp = _tc_layer(pair(p), normw, W1, CA, CB)
    p = _prop(CB, flat(gp), srci, dsti)
    m = _tc_mid(pair(p), normw, CB)
    p = _prop(CB, flat(m), srci, dsti)
    gp = _tc_layer2(pair(p), normw, W2, W3, CB, CA)
    p = _prop(CA, flat(gp), srci, dsti)
    m = _tc_mid(pair(p), normw, CA)
    p = _prop(CA, flat(m), srci, dsti)
    ot, ob = _tc_final(pair(p), normw)
    return jnp.concatenate([ot, ob[:N - HALF]], axis=0)


# speedup vs baseline: 1.6172x; 1.0333x over previous
import functools

import jax
import jax.numpy as jnp
from jax import lax
from jax.experimental import pallas as pl
from jax.experimental.pallas import tpu as pltpu
from jax.experimental.pallas import tpu_sc as plsc

N = 10000
E = 160000
IN_FEATS = 256
N_HIDDEN = 512
N_CLASSES = 256

NC = 2
NS = 16
NW = NC * NS
EPW = E // NS
CHUNK = 128
NCH = 80
EPW_PAD = NCH * CHUNK
PADE = EPW_PAD - EPW
NPAD = 10240
HALF = NPAD // 2
STRIPE = NPAD // NS
F = 64
BP = 512
NBUF = 5


def _sc_mesh():
    return plsc.VectorSubcoreMesh(core_axis_name="c", subcore_axis_name="s")


def _deg_body(dstidx, degp, dst_v, ones_v, zbuf, acc):
    core = lax.axis_index("c")
    sub = lax.axis_index("s")
    pltpu.sync_copy(dstidx.at[sub], dst_v)

    def _fill(i, _):
        ones_v[pl.ds(i * 16, 16)] = jnp.full((16,), 1.0, jnp.float32)
        return _

    def _zero(i, _):
        zbuf[pl.ds(i * 16, 16)] = jnp.zeros((16,), jnp.float32)
        return _

    lax.fori_loop(0, CHUNK // 16, _fill, None)
    lax.fori_loop(0, STRIPE // 16, _zero, None)
    pltpu.sync_copy(zbuf, acc.at[pl.ds(sub * STRIPE, STRIPE)])
    plsc.subcore_barrier()

    def _scat(j, _):
        pltpu.sync_copy(ones_v, acc.at[dst_v.at[j]], add=True)
        return _

    half = NCH // NC
    lax.fori_loop(core * half, (core + 1) * half, _scat, None)
    plsc.subcore_barrier()
    for k in range(NC):
        @pl.when(core == k)
        def _(k=k):
            pltpu.sync_copy(acc.at[pl.ds(sub * STRIPE, STRIPE)],
                            degp.at[k, pl.ds(sub * STRIPE, STRIPE)])


def _deg_kernel(dstidx):
    return pl.kernel(
        _deg_body,
        out_type=jax.ShapeDtypeStruct((NC, NPAD), jnp.float32),
        mesh=_sc_mesh(),
        compiler_params=pltpu.CompilerParams(use_tc_tiling_on_sc=False),
        scratch_types=[
            pltpu.VMEM((NCH, CHUNK), jnp.int32),
            pltpu.VMEM((CHUNK,), jnp.float32),
            pltpu.VMEM((STRIPE,), jnp.float32),
            pltpu.VMEM_SHARED((NPAD,), jnp.float32),
        ],
    )(dstidx)


def _prop_body(C, g2, srcidx, dstidx, out, src_v, dst_v, bufs, zbuf, acc, sem):
    core = lax.axis_index("c")
    sub = lax.axis_index("s")
    pltpu.sync_copy(srcidx.at[sub], src_v)
    pltpu.sync_copy(dstidx.at[sub], dst_v)

    GPR = F // 16

    def _zb(i, _):
        zbuf[i // GPR, pl.ds((i % GPR) * 16, 16)] = jnp.zeros((16,), jnp.float32)
        return _

    lax.fori_loop(0, 160 * GPR, _zb, None)

    for q in range(4):
        pltpu.sync_copy(zbuf, acc.at[pl.ds(sub * STRIPE + q * 160, 160)])
    plsc.subcore_barrier()

    CH = C // NC
    cbase = core * CH
    for cc in range(CH):
        def _grp(t, _):
            j0 = t * NBUF
            gds = [pltpu.async_copy(g2.at[cbase + cc].at[src_v.at[j0 + b]],
                                    bufs.at[b], sem.at[b])
                   for b in range(NBUF)]
            sds = []
            for b in range(NBUF):
                gds[b].wait()
                sds.append(pltpu.async_copy(bufs.at[b],
                                            acc.at[dst_v.at[j0 + b]],
                                            sem.at[b], add=True))
            for sd in sds:
                sd.wait()
            return _

        lax.fori_loop(0, NCH // NBUF, _grp, None)
        plsc.subcore_barrier()
        pltpu.sync_copy(acc.at[pl.ds(sub * STRIPE, STRIPE)],
                        out.at[cbase + cc, pl.ds(sub * STRIPE, STRIPE)])
        if cc + 1 < CH:
            for q in range(4):
                pltpu.sync_copy(zbuf, acc.at[pl.ds(sub * STRIPE + q * 160, 160)])
        plsc.subcore_barrier()


def _prop(C, g2, srcidx, dstidx):
    return pl.kernel(
        functools.partial(_prop_body, C),
        out_type=jax.ShapeDtypeStruct((C, NPAD, F), jnp.float32),
        mesh=_sc_mesh(),
        compiler_params=pltpu.CompilerParams(use_tc_tiling_on_sc=False),
        scratch_types=[
            pltpu.VMEM((NCH, CHUNK), jnp.int32),
            pltpu.VMEM((NCH, CHUNK), jnp.int32),
            pltpu.VMEM((NBUF, CHUNK, F), jnp.float32),
            pltpu.VMEM((160, F), jnp.float32),
            pltpu.VMEM_SHARED((NPAD, F), jnp.float32),
            pltpu.SemaphoreType.DMA((NBUF,)),
        ],
    )(g2, srcidx, dstidx)


def _ta_body(degp_ref, xt_ref, xb_ref, gp_ref, normw_ref):
    deg = degp_ref[0] + degp_ref[1]
    nrm = lax.rsqrt(jnp.maximum(deg, 1.0))
    nt, nb = nrm[:, 0:1], nrm[:, 1:2]
    normw = jnp.concatenate(
        [jnp.broadcast_to(nt, (BP, F)), jnp.broadcast_to(nb, (BP, F))], axis=1)
    normw_ref[...] = normw
    st = xt_ref[...] * nt
    sb = xb_ref[...] * nb
    for c in range(IN_FEATS // F):
        gp_ref[c] = jnp.concatenate(
            [st[:, c * F:(c + 1) * F], sb[:, c * F:(c + 1) * F]], axis=1)


def _tc_prescale(degs, featpad):
    CI = IN_FEATS // F
    return pl.pallas_call(
        _ta_body,
        grid=(HALF // BP,),
        in_specs=[
            pl.BlockSpec((NC, BP, 2), lambda i: (0, i, 0)),
            pl.BlockSpec((BP, IN_FEATS), lambda i: (i, 0)),
            pl.BlockSpec((BP, IN_FEATS), lambda i: (HALF // BP + i, 0)),
        ],
        out_specs=[
            pl.BlockSpec((CI, BP, 2 * F), lambda i: (0, i, 0)),
            pl.BlockSpec((BP, 2 * F), lambda i: (i, 0)),
        ],
        out_shape=[
            jax.ShapeDtypeStruct((CI, HALF, 2 * F), jnp.float32),
            jax.ShapeDtypeStruct((HALF, 2 * F), jnp.float32),
        ],
    )(degs, featpad, featpad)


def _tb_body(pp_ref, normw_ref, m_ref):
    nw = normw_ref[...]
    m_ref[0] = pp_ref[0] * (nw * nw)


def _tc_mid(pp, normw, C):
    return pl.pallas_call(
        _tb_body,
        grid=(C, HALF // BP),
        in_specs=[
            pl.BlockSpec((1, BP, 2 * F), lambda c, i: (c, i, 0)),
            pl.BlockSpec((BP, 2 * F), lambda c, i: (i, 0)),
        ],
        out_specs=pl.BlockSpec((1, BP, 2 * F), lambda c, i: (c, i, 0)),
        out_shape=jax.ShapeDtypeStruct((C, HALF, 2 * F), jnp.float32),
    )(pp, normw)


def _dotT(x, w):
    return lax.dot_general(x, w, (((1,), (1,)), ((), ())),
                           preferred_element_type=jnp.float32)


def _tc_layer_body(CI, CO, pp_ref, normw_ref, w_ref, gp_ref):
    nw = normw_ref[...]
    dout = w_ref.shape[0]
    acct = jnp.zeros((BP, dout), jnp.float32)
    accb = jnp.zeros((BP, dout), jnp.float32)
    for c in range(CI):
        t = pp_ref[c] * nw
        wc = w_ref[:, c * F:(c + 1) * F]
        acct = acct + _dotT(t[:, :F], wc)
        accb = accb + _dotT(t[:, F:], wc)
    ht = jnp.maximum(acct, 0.0) * nw[:, 0:1]
    hb = jnp.maximum(accb, 0.0) * nw[:, F:F + 1]
    for co in range(CO):
        gp_ref[co] = jnp.concatenate(
            [ht[:, co * F:(co + 1) * F], hb[:, co * F:(co + 1) * F]], axis=1)


def _tc_layer(pp, normw, W, CI, CO):
    return pl.pallas_call(
        functools.partial(_tc_layer_body, CI, CO),
        grid=(HALF // BP,),
        in_specs=[
            pl.BlockSpec((CI, BP, 2 * F), lambda i: (0, i, 0)),
            pl.BlockSpec((BP, 2 * F), lambda i: (i, 0)),
            pl.BlockSpec(W.shape, lambda i: (0, 0)),
        ],
        out_specs=pl.BlockSpec((CO, BP, 2 * F), lambda i: (0, i, 0)),
        out_shape=jax.ShapeDtypeStruct((CO, HALF, 2 * F), jnp.float32),
    )(pp, normw, W)


def _tc_layer2_body(CI, CO, pp_ref, normw_ref, w2_ref, w3_ref, gp_ref):
    nw = normw_ref[...]
    acct = jnp.zeros((BP, N_HIDDEN), jnp.float32)
    accb = jnp.zeros((BP, N_HIDDEN), jnp.float32)
    for c in range(CI):
        t = pp_ref[c] * nw
        wc = w2_ref[:, c * F:(c + 1) * F]
        acct = acct + _dotT(t[:, :F], wc)
        accb = accb + _dotT(t[:, F:], wc)
    zt = _dotT(jnp.maximum(acct, 0.0), w3_ref[...]) * nw[:, 0:1]
    zb = _dotT(jnp.maximum(accb, 0.0), w3_ref[...]) * nw[:, F:F + 1]
    for co in range(CO):
        gp_ref[co] = jnp.concatenate(
            [zt[:, co * F:(co + 1) * F], zb[:, co * F:(co + 1) * F]], axis=1)


def _tc_layer2(pp, normw, W2, W3, CI, CO):
    return pl.pallas_call(
        functools.partial(_tc_layer2_body, CI, CO),
        grid=(HALF // BP,),
        in_specs=[
            pl.BlockSpec((CI, BP, 2 * F), lambda i: (0, i, 0)),
            pl.BlockSpec((BP, 2 * F), lambda i: (i, 0)),
            pl.BlockSpec(W2.shape, lambda i: (0, 0)),
            pl.BlockSpec(W3.shape, lambda i: (0, 0)),
        ],
        out_specs=pl.BlockSpec((CO, BP, 2 * F), lambda i: (0, i, 0)),
        out_shape=jax.ShapeDtypeStruct((CO, HALF, 2 * F), jnp.float32),
    )(pp, normw, W2, W3)


def _td_body(pp_ref, normw_ref, ot_ref, ob_ref):
    nw = normw_ref[...]
    ts, bs = [], []
    for c in range(N_CLASSES // F):
        t = pp_ref[c] * nw
        ts.append(t[:, :F])
        bs.append(t[:, F:])
    ot_ref[...] = jnp.concatenate(ts, axis=1)
    ob_ref[...] = jnp.concatenate(bs, axis=1)


def _tc_final(pp, normw):
    CI = N_CLASSES // F
    return pl.pallas_call(
        _td_body,
        grid=(HALF // BP,),
        in_specs=[
            pl.BlockSpec((CI, BP, 2 * F), lambda i: (0, i, 0)),
            pl.BlockSpec((BP, 2 * F), lambda i: (i, 0)),
        ],
        out_specs=[
            pl.BlockSpec((BP, N_CLASSES), lambda i: (i, 0)),
            pl.BlockSpec((BP, N_CLASSES), lambda i: (i, 0)),
        ],
        out_shape=[
            jax.ShapeDtypeStruct((HALF, N_CLASSES), jnp.float32),
            jax.ShapeDtypeStruct((HALF, N_CLASSES), jnp.float32),
        ],
    )(pp, normw)


def kernel(features, edge_index, W1, W2, W3):
    src = edge_index[0]
    dst = edge_index[1]

    def sig(v):
        return jnp.where(v < HALF, 2 * v, 2 * v - (2 * HALF - 1))

    w = jnp.arange(NS, dtype=jnp.int32)[:, None]
    i = jnp.arange(PADE, dtype=jnp.int32)[None, :]
    pad_src = (w * 997 + i * 131) % N
    pad_dst = (NPAD - 1 - 2 * (i % 8)) + jnp.zeros((NS, 1), jnp.int32)
    srcp = jnp.concatenate([sig(src).reshape(NS, EPW), pad_src], axis=1)
    dstp = jnp.concatenate([sig(dst).reshape(NS, EPW), pad_dst], axis=1)
    srci = srcp.reshape(NS, NCH, CHUNK)
    dsti = dstp.reshape(NS, NCH, CHUNK)

    degp = _deg_kernel(dsti)
    degs = degp.reshape(NC, HALF, 2)
    featpad = jnp.pad(features, ((0, NPAD - N), (0, 0)))

    CA = IN_FEATS // F
    CB = N_HIDDEN // F

    def pair(p):
        return p.reshape(p.shape[0], HALF, 2 * F)

    def flat(gp):
        return gp.reshape(gp.shape[0], NPAD, F)

    gp, normw = _tc_prescale(degs, featpad)
    p = _prop(CA, flat(gp), srci, dsti)
    m = _tc_mid(pair(p), normw, CA)
    p = _prop(CA, flat(m), srci, dsti)
    gp = _tc_layer(pair(p), normw, W1, CA, CB)
    p = _prop(CB, flat(gp), srci, dsti)
    m = _tc_mid(pair(p), normw, CB)
    p = _prop(CB, flat(m), srci, dsti)
    gp = _tc_layer2(pair(p), normw, W2, W3, CB, CA)
    p = _prop(CA, flat(gp), srci, dsti)
    m = _tc_mid(pair(p), normw, CA)
    p = _prop(CA, flat(m), srci, dsti)
    ot, ob = _tc_final(pair(p), normw)
    return jnp.concatenate([ot, ob[:N - HALF]], axis=0)


# speedup vs baseline: 1.6920x; 1.0463x over previous
import functools

import jax
import jax.numpy as jnp
from jax import lax
from jax.experimental import pallas as pl
from jax.experimental.pallas import tpu as pltpu
from jax.experimental.pallas import tpu_sc as plsc

N = 10000
E = 160000
IN_FEATS = 256
N_HIDDEN = 512
N_CLASSES = 256

NC = 2
NS = 16
NW = NC * NS
EPW = E // NS
CHUNK = 128
NCH = 80
EPW_PAD = NCH * CHUNK
PADE = EPW_PAD - EPW
NPAD = 10240
HALF = NPAD // 2
STRIPE = NPAD // NS
F = 64
BP = 1024
NBUF = 5


def _sc_mesh():
    return plsc.VectorSubcoreMesh(core_axis_name="c", subcore_axis_name="s")


def _deg_body(dstidx, degp, dst_v, ones_v, zbuf, acc):
    core = lax.axis_index("c")
    sub = lax.axis_index("s")
    pltpu.sync_copy(dstidx.at[sub], dst_v)

    def _fill(i, _):
        ones_v[pl.ds(i * 16, 16)] = jnp.full((16,), 1.0, jnp.float32)
        return _

    def _zero(i, _):
        zbuf[pl.ds(i * 16, 16)] = jnp.zeros((16,), jnp.float32)
        return _

    lax.fori_loop(0, CHUNK // 16, _fill, None)
    lax.fori_loop(0, STRIPE // 16, _zero, None)
    pltpu.sync_copy(zbuf, acc.at[pl.ds(sub * STRIPE, STRIPE)])
    plsc.subcore_barrier()

    def _scat(j, _):
        pltpu.sync_copy(ones_v, acc.at[dst_v.at[j]], add=True)
        return _

    half = NCH // NC
    lax.fori_loop(core * half, (core + 1) * half, _scat, None)
    plsc.subcore_barrier()
    for k in range(NC):
        @pl.when(core == k)
        def _(k=k):
            pltpu.sync_copy(acc.at[pl.ds(sub * STRIPE, STRIPE)],
                            degp.at[k, pl.ds(sub * STRIPE, STRIPE)])


def _deg_kernel(dstidx):
    return pl.kernel(
        _deg_body,
        out_type=jax.ShapeDtypeStruct((NC, NPAD), jnp.float32),
        mesh=_sc_mesh(),
        compiler_params=pltpu.CompilerParams(use_tc_tiling_on_sc=False),
        scratch_types=[
            pltpu.VMEM((NCH, CHUNK), jnp.int32),
            pltpu.VMEM((CHUNK,), jnp.float32),
            pltpu.VMEM((STRIPE,), jnp.float32),
            pltpu.VMEM_SHARED((NPAD,), jnp.float32),
        ],
    )(dstidx)


def _prop_body(C, g2, srcidx, dstidx, out, src_v, dst_v, bufs, zbuf, acc, sem):
    core = lax.axis_index("c")
    sub = lax.axis_index("s")
    pltpu.sync_copy(srcidx.at[sub], src_v)
    pltpu.sync_copy(dstidx.at[sub], dst_v)

    GPR = F // 16

    def _zb(i, _):
        zbuf[i // GPR, pl.ds((i % GPR) * 16, 16)] = jnp.zeros((16,), jnp.float32)
        return _

    lax.fori_loop(0, 160 * GPR, _zb, None)

    for q in range(4):
        pltpu.sync_copy(zbuf, acc.at[pl.ds(sub * STRIPE + q * 160, 160)])
    plsc.subcore_barrier()

    CH = C // NC
    cbase = core * CH
    for cc in range(CH):
        def _grp(t, _):
            j0 = t * NBUF
            gds = [pltpu.async_copy(g2.at[cbase + cc].at[src_v.at[j0 + b]],
                                    bufs.at[b], sem.at[b])
                   for b in range(NBUF)]
            sds = []
            for b in range(NBUF):
                gds[b].wait()
                sds.append(pltpu.async_copy(bufs.at[b],
                                            acc.at[dst_v.at[j0 + b]],
                                            sem.at[b], add=True))
            for sd in sds:
                sd.wait()
            return _

        lax.fori_loop(0, NCH // NBUF, _grp, None)
        plsc.subcore_barrier()
        pltpu.sync_copy(acc.at[pl.ds(sub * STRIPE, STRIPE)],
                        out.at[cbase + cc, pl.ds(sub * STRIPE, STRIPE)])
        if cc + 1 < CH:
            for q in range(4):
                pltpu.sync_copy(zbuf, acc.at[pl.ds(sub * STRIPE + q * 160, 160)])
        plsc.subcore_barrier()


def _prop(C, g2, srcidx, dstidx):
    return pl.kernel(
        functools.partial(_prop_body, C),
        out_type=jax.ShapeDtypeStruct((C, NPAD, F), jnp.float32),
        mesh=_sc_mesh(),
        compiler_params=pltpu.CompilerParams(use_tc_tiling_on_sc=False),
        scratch_types=[
            pltpu.VMEM((NCH, CHUNK), jnp.int32),
            pltpu.VMEM((NCH, CHUNK), jnp.int32),
            pltpu.VMEM((NBUF, CHUNK, F), jnp.float32),
            pltpu.VMEM((160, F), jnp.float32),
            pltpu.VMEM_SHARED((NPAD, F), jnp.float32),
            pltpu.SemaphoreType.DMA((NBUF,)),
        ],
    )(g2, srcidx, dstidx)


def _ta_body(degp_ref, xt_ref, xb_ref, gp_ref, normw_ref):
    deg = degp_ref[0] + degp_ref[1]
    nrm = lax.rsqrt(jnp.maximum(deg, 1.0))
    nt, nb = nrm[:, 0:1], nrm[:, 1:2]
    normw = jnp.concatenate(
        [jnp.broadcast_to(nt, (BP, F)), jnp.broadcast_to(nb, (BP, F))], axis=1)
    normw_ref[...] = normw
    st = xt_ref[...] * nt
    sb = xb_ref[...] * nb
    for c in range(IN_FEATS // F):
        gp_ref[c] = jnp.concatenate(
            [st[:, c * F:(c + 1) * F], sb[:, c * F:(c + 1) * F]], axis=1)


def _tc_prescale(degs, featpad):
    CI = IN_FEATS // F
    return pl.pallas_call(
        _ta_body,
        grid=(HALF // BP,),
        in_specs=[
            pl.BlockSpec((NC, BP, 2), lambda i: (0, i, 0)),
            pl.BlockSpec((BP, IN_FEATS), lambda i: (i, 0)),
            pl.BlockSpec((BP, IN_FEATS), lambda i: (HALF // BP + i, 0)),
        ],
        out_specs=[
            pl.BlockSpec((CI, BP, 2 * F), lambda i: (0, i, 0)),
            pl.BlockSpec((BP, 2 * F), lambda i: (i, 0)),
        ],
        out_shape=[
            jax.ShapeDtypeStruct((CI, HALF, 2 * F), jnp.float32),
            jax.ShapeDtypeStruct((HALF, 2 * F), jnp.float32),
        ],
    )(degs, featpad, featpad)


def _tb_body(pp_ref, normw_ref, m_ref):
    nw = normw_ref[...]
    m_ref[0] = pp_ref[0] * (nw * nw)


def _tc_mid(pp, normw, C):
    return pl.pallas_call(
        _tb_body,
        grid=(C, HALF // BP),
        in_specs=[
            pl.BlockSpec((1, BP, 2 * F), lambda c, i: (c, i, 0)),
            pl.BlockSpec((BP, 2 * F), lambda c, i: (i, 0)),
        ],
        out_specs=pl.BlockSpec((1, BP, 2 * F), lambda c, i: (c, i, 0)),
        out_shape=jax.ShapeDtypeStruct((C, HALF, 2 * F), jnp.float32),
    )(pp, normw)


def _dotT(x, w):
    return lax.dot_general(x, w, (((1,), (1,)), ((), ())),
                           preferred_element_type=jnp.float32)


def _tc_layer_body(CI, CO, pp_ref, normw_ref, w_ref, gp_ref):
    nw = normw_ref[...]
    ts = [pp_ref[c] * nw for c in range(CI)]
    tt = jnp.concatenate([t[:, :F] for t in ts], axis=1)
    tb = jnp.concatenate([t[:, F:] for t in ts], axis=1)
    ht = jnp.maximum(_dotT(tt, w_ref[...]), 0.0) * nw[:, 0:1]
    hb = jnp.maximum(_dotT(tb, w_ref[...]), 0.0) * nw[:, F:F + 1]
    for co in range(CO):
        gp_ref[co] = jnp.concatenate(
            [ht[:, co * F:(co + 1) * F], hb[:, co * F:(co + 1) * F]], axis=1)


def _tc_layer(pp, normw, W, CI, CO):
    return pl.pallas_call(
        functools.partial(_tc_layer_body, CI, CO),
        grid=(HALF // BP,),
        in_specs=[
            pl.BlockSpec((CI, BP, 2 * F), lambda i: (0, i, 0)),
            pl.BlockSpec((BP, 2 * F), lambda i: (i, 0)),
            pl.BlockSpec(W.shape, lambda i: (0, 0)),
        ],
        out_specs=pl.BlockSpec((CO, BP, 2 * F), lambda i: (0, i, 0)),
        out_shape=jax.ShapeDtypeStruct((CO, HALF, 2 * F), jnp.float32),
    )(pp, normw, W)


def _tc_layer2_body(CI, CO, pp_ref, normw_ref, w2_ref, w3_ref, gp_ref):
    nw = normw_ref[...]
    ts = [pp_ref[c] * nw for c in range(CI)]
    tt = jnp.concatenate([t[:, :F] for t in ts], axis=1)
    tb = jnp.concatenate([t[:, F:] for t in ts], axis=1)
    zt = _dotT(jnp.maximum(_dotT(tt, w2_ref[...]), 0.0), w3_ref[...]) * nw[:, 0:1]
    zb = _dotT(jnp.maximum(_dotT(tb, w2_ref[...]), 0.0), w3_ref[...]) * nw[:, F:F + 1]
    for co in range(CO):
        gp_ref[co] = jnp.concatenate(
            [zt[:, co * F:(co + 1) * F], zb[:, co * F:(co + 1) * F]], axis=1)


def _tc_layer2(pp, normw, W2, W3, CI, CO):
    return pl.pallas_call(
        functools.partial(_tc_layer2_body, CI, CO),
        grid=(HALF // BP,),
        in_specs=[
            pl.BlockSpec((CI, BP, 2 * F), lambda i: (0, i, 0)),
            pl.BlockSpec((BP, 2 * F), lambda i: (i, 0)),
            pl.BlockSpec(W2.shape, lambda i: (0, 0)),
            pl.BlockSpec(W3.shape, lambda i: (0, 0)),
        ],
        out_specs=pl.BlockSpec((CO, BP, 2 * F), lambda i: (0, i, 0)),
        out_shape=jax.ShapeDtypeStruct((CO, HALF, 2 * F), jnp.float32),
    )(pp, normw, W2, W3)


def _td_body(pp_ref, normw_ref, ot_ref, ob_ref):
    nw = normw_ref[...]
    ts, bs = [], []
    for c in range(N_CLASSES // F):
        t = pp_ref[c] * nw
        ts.append(t[:, :F])
        bs.append(t[:, F:])
    ot_ref[...] = jnp.concatenate(ts, axis=1)
    ob_ref[...] = jnp.concatenate(bs, axis=1)


def _tc_final(pp, normw):
    CI = N_CLASSES // F
    return pl.pallas_call(
        _td_body,
        grid=(HALF // BP,),
        in_specs=[
            pl.BlockSpec((CI, BP, 2 * F), lambda i: (0, i, 0)),
            pl.BlockSpec((BP, 2 * F), lambda i: (i, 0)),
        ],
        out_specs=[
            pl.BlockSpec((BP, N_CLASSES), lambda i: (i, 0)),
            pl.BlockSpec((BP, N_CLASSES), lambda i: (i, 0)),
        ],
        out_shape=[
            jax.ShapeDtypeStruct((HALF, N_CLASSES), jnp.float32),
            jax.ShapeDtypeStruct((HALF, N_CLASSES), jnp.float32),
        ],
    )(pp, normw)


def kernel(features, edge_index, W1, W2, W3):
    src = edge_index[0]
    dst = edge_index[1]

    def sig(v):
        return jnp.where(v < HALF, 2 * v, 2 * v - (2 * HALF - 1))

    w = jnp.arange(NS, dtype=jnp.int32)[:, None]
    i = jnp.arange(PADE, dtype=jnp.int32)[None, :]
    pad_src = (w * 997 + i * 131) % N
    pad_dst = (NPAD - 1 - 2 * (i % 8)) + jnp.zeros((NS, 1), jnp.int32)
    srcp = jnp.concatenate([sig(src).reshape(NS, EPW), pad_src], axis=1)
    dstp = jnp.concatenate([sig(dst).reshape(NS, EPW), pad_dst], axis=1)
    srci = srcp.reshape(NS, NCH, CHUNK)
    dsti = dstp.reshape(NS, NCH, CHUNK)

    degp = _deg_kernel(dsti)
    degs = degp.reshape(NC, HALF, 2)
    featpad = jnp.pad(features, ((0, NPAD - N), (0, 0)))

    CA = IN_FEATS // F
    CB = N_HIDDEN // F

    def pair(p):
        return p.reshape(p.shape[0], HALF, 2 * F)

    def flat(gp):
        return gp.reshape(gp.shape[0], NPAD, F)

    gp, normw = _tc_prescale(degs, featpad)
    p = _prop(CA, flat(gp), srci, dsti)
    m = _tc_mid(pair(p), normw, CA)
    p = _prop(CA, flat(m), srci, dsti)
    gp = _tc_layer(pair(p), normw, W1, CA, CB)
    p = _prop(CB, flat(gp), srci, dsti)
    m = _tc_mid(pair(p), normw, CB)
    p = _prop(CB, flat(m), srci, dsti)
    gp = _tc_layer2(pair(p), normw, W2, W3, CB, CA)
    p = _prop(CA, flat(gp), srci, dsti)
    m = _tc_mid(pair(p), normw, CA)
    p = _prop(CA, flat(m), srci, dsti)
    ot, ob = _tc_final(pair(p), normw)
    return jnp.concatenate([ot, ob[:N - HALF]], axis=0)
